# baseline (speedup 1.0000x reference)
import functools

import jax
import jax.numpy as jnp
from jax import lax
from jax.experimental import pallas as pl
from jax.experimental.pallas import tpu as pltpu
from jax.experimental.pallas import tpu_sc as plsc

N_NODES = 10000
N_EDGES = 320000
D = 128

NC = 2
NS = 16
NW = NC * NS

CHUNK = 128
E_PAD = 327680
CHUNKS = E_PAD // (NW * CHUNK)
HALF = CHUNKS // 2
N_PAD = 10240
ROWS_PER_TILE = N_PAD // NS

_mesh = plsc.VectorSubcoreMesh(core_axis_name="c", subcore_axis_name="s")


@functools.partial(
    pl.kernel,
    mesh=_mesh,
    out_type=jax.ShapeDtypeStruct((NC, N_PAD, D), jnp.float32),
    scratch_types=[
        pltpu.VMEM((HALF, CHUNK), jnp.int32),
        pltpu.VMEM((HALF, CHUNK), jnp.int32),
        pltpu.VMEM((2, CHUNK, D), jnp.float32),
        pltpu.VMEM_SHARED((N_PAD, D), jnp.float32),
        pltpu.SemaphoreType.DMA,
        pltpu.SemaphoreType.DMA,
        pltpu.SemaphoreType.DMA,
        pltpu.SemaphoreType.DMA,
    ],
)
def _sc_agg(src_hbm, dst_hbm, x_hbm, out_hbm, src_v, dst_v, rows_v, acc_sh,
            gsem0, gsem1, ssem0, ssem1):
    cid = lax.axis_index("c")
    sid = lax.axis_index("s")
    wid = sid * NC + cid

    pltpu.async_copy(src_hbm.at[wid, pl.ds(0, HALF)], src_v, ssem0)
    pltpu.async_copy(dst_hbm.at[wid, pl.ds(0, HALF)], dst_v, ssem1)

    zval = jnp.zeros((16,), jnp.float32)

    def zstore(r, carry):
        for cc in range(D // 16):
            rows_v[0, r, pl.ds(cc * 16, 16)] = zval
        return carry

    lax.fori_loop(0, CHUNK, zstore, 0)
    zbase = sid * ROWS_PER_TILE
    for k in range(ROWS_PER_TILE // CHUNK):
        pltpu.async_copy(rows_v.at[0],
                         acc_sh.at[pl.ds(zbase + k * CHUNK, CHUNK)], gsem0)
    for k in range(ROWS_PER_TILE // CHUNK):
        pltpu.make_async_copy(rows_v.at[0],
                              acc_sh.at[pl.ds(zbase + k * CHUNK, CHUNK)],
                              gsem0).wait()
    pltpu.make_async_copy(src_hbm.at[wid, pl.ds(0, HALF)], src_v, ssem0).wait()
    pltpu.make_async_copy(dst_hbm.at[wid, pl.ds(0, HALF)], dst_v, ssem1).wait()

    plsc.subcore_barrier()

    gsems = (gsem0, gsem1)

    def fire_g(j, p):
        pltpu.async_copy(x_hbm.at[src_v.at[j]], rows_v.at[p], gsems[p])

    def wait_g(j, p):
        pltpu.make_async_copy(x_hbm.at[src_v.at[j]], rows_v.at[p], gsems[p]).wait()

    def scat(j, p):
        pltpu.sync_copy(rows_v.at[p], acc_sh.at[dst_v.at[j]], add=True)

    for h in range(2):
        if h == 1:
            pltpu.sync_copy(src_hbm.at[wid, pl.ds(HALF, HALF)], src_v)
            pltpu.sync_copy(dst_hbm.at[wid, pl.ds(HALF, HALF)], dst_v)

        fire_g(0, 0)

        def body(i, carry):
            j = 2 * i
            fire_g(j + 1, 1)
            wait_g(j, 0)
            scat(j, 0)
            fire_g(j + 2, 0)
            wait_g(j + 1, 1)
            scat(j + 1, 1)
            return carry

        lax.fori_loop(0, HALF // 2 - 1, body, 0)
        fire_g(HALF - 1, 1)
        wait_g(HALF - 2, 0)
        scat(HALF - 2, 0)
        wait_g(HALF - 1, 1)
        scat(HALF - 1, 1)

    plsc.subcore_barrier()

    pltpu.sync_copy(
        acc_sh.at[pl.ds(zbase, ROWS_PER_TILE)],
        out_hbm.at[cid, pl.ds(zbase, ROWS_PER_TILE)],
    )


_TC_BLOCK = 2000
_ROW_SPEC = pl.BlockSpec((_TC_BLOCK, D), lambda i: (i, 0))
_W_SPEC = pl.BlockSpec((D, D), lambda i: (0, 0))
_B_SPEC = pl.BlockSpec((1, D), lambda i: (0, 0))
_GRID = (N_NODES // _TC_BLOCK,)
_OUT_SHAPE = jax.ShapeDtypeStruct((N_NODES, D), jnp.float32)


def _tc_self_body(x_ref, wt_ref, b_ref, o_ref):
    o_ref[...] = (
        jnp.dot(x_ref[...], wt_ref[...], preferred_element_type=jnp.float32)
        + b_ref[...]
    )


def _tc_self(x, wt, b2d):
    return pl.pallas_call(
        _tc_self_body,
        grid=_GRID,
        in_specs=[_ROW_SPEC, _W_SPEC, _B_SPEC],
        out_specs=_ROW_SPEC,
        out_shape=_OUT_SHAPE,
    )(x, wt, b2d)


def _tc_agg_body(y_ref, a0_ref, a1_ref, wt_ref, o_ref):
    agg = a0_ref[...] + a1_ref[...]
    o_ref[...] = y_ref[...] + jnp.dot(
        agg, wt_ref[...], preferred_element_type=jnp.float32)


def _tc_agg_linear(y0, a0, a1, wt):
    return pl.pallas_call(
        _tc_agg_body,
        grid=_GRID,
        in_specs=[_ROW_SPEC, _ROW_SPEC, _ROW_SPEC, _W_SPEC],
        out_specs=_ROW_SPEC,
        out_shape=_OUT_SHAPE,
    )(y0, a0, a1, wt)


def kernel(x, edge_index, W, b):
    src = edge_index[0].astype(jnp.int32)
    dst = edge_index[1].astype(jnp.int32)
    per_w = N_EDGES // NW
    pad_w = CHUNKS * CHUNK - per_w
    pad_dst_row = N_NODES + jnp.arange(pad_w, dtype=jnp.int32)
    pad_src_row = (jnp.arange(pad_w, dtype=jnp.int32) * 41) % N_NODES
    src_p = jnp.concatenate(
        [src.reshape(NW, per_w), jnp.broadcast_to(pad_src_row, (NW, pad_w))], axis=1)
    dst_p = jnp.concatenate(
        [dst.reshape(NW, per_w), jnp.broadcast_to(pad_dst_row, (NW, pad_w))], axis=1)
    src_p = src_p.reshape(NW, CHUNKS, CHUNK)
    dst_p = dst_p.reshape(NW, CHUNKS, CHUNK)
    wt = W.T
    y0 = _tc_self(x, wt, b.reshape(1, D))
    agg = _sc_agg(src_p, dst_p, x)
    return _tc_agg_linear(y0, agg[0], agg[1], wt)


# speedup vs baseline: 1.0057x; 1.0057x over previous
import functools

import jax
import jax.numpy as jnp
from jax import lax
from jax.experimental import pallas as pl
from jax.experimental.pallas import tpu as pltpu
from jax.experimental.pallas import tpu_sc as plsc

N_NODES = 10000
N_EDGES = 320000
D = 128

NC = 2
NS = 16
NW = NC * NS

CHUNK = 128
E_PAD = 327680
CHUNKS = E_PAD // (NW * CHUNK)
HALF = CHUNKS // 2
N_PAD = 10240
ROWS_PER_TILE = N_PAD // NS

_mesh = plsc.VectorSubcoreMesh(core_axis_name="c", subcore_axis_name="s")


@functools.partial(
    pl.kernel,
    mesh=_mesh,
    out_type=jax.ShapeDtypeStruct((NC, N_PAD, D), jnp.float32),
    scratch_types=[
        pltpu.VMEM((HALF, CHUNK), jnp.int32),
        pltpu.VMEM((HALF, CHUNK), jnp.int32),
        pltpu.VMEM((2, CHUNK, D), jnp.float32),
        pltpu.VMEM_SHARED((N_PAD, D), jnp.float32),
        pltpu.SemaphoreType.DMA,
        pltpu.SemaphoreType.DMA,
        pltpu.SemaphoreType.DMA,
        pltpu.SemaphoreType.DMA,
    ],
)
def _sc_agg(src_hbm, dst_hbm, x_hbm, out_hbm, src_v, dst_v, rows_v, acc_sh,
            gsem0, gsem1, ssem0, ssem1):
    cid = lax.axis_index("c")
    sid = lax.axis_index("s")
    wid = sid * NC + cid

    pltpu.async_copy(src_hbm.at[wid, pl.ds(0, HALF)], src_v, ssem0)
    pltpu.async_copy(dst_hbm.at[wid, pl.ds(0, HALF)], dst_v, ssem1)

    zval = jnp.zeros((16,), jnp.float32)

    def zstore(r, carry):
        for cc in range(D // 16):
            rows_v[0, r, pl.ds(cc * 16, 16)] = zval
        return carry

    lax.fori_loop(0, CHUNK, zstore, 0)
    zbase = sid * ROWS_PER_TILE
    for k in range(ROWS_PER_TILE // CHUNK):
        pltpu.async_copy(rows_v.at[0],
                         acc_sh.at[pl.ds(zbase + k * CHUNK, CHUNK)], gsem0)
    for k in range(ROWS_PER_TILE // CHUNK):
        pltpu.make_async_copy(rows_v.at[0],
                              acc_sh.at[pl.ds(zbase + k * CHUNK, CHUNK)],
                              gsem0).wait()
    pltpu.make_async_copy(src_hbm.at[wid, pl.ds(0, HALF)], src_v, ssem0).wait()
    pltpu.make_async_copy(dst_hbm.at[wid, pl.ds(0, HALF)], dst_v, ssem1).wait()

    plsc.subcore_barrier()

    gsems = (gsem0, gsem1)

    def fire_g(j, p):
        pltpu.async_copy(x_hbm.at[src_v.at[j]], rows_v.at[p], gsems[p])

    def wait_g(j, p):
        pltpu.make_async_copy(x_hbm.at[src_v.at[j]], rows_v.at[p], gsems[p]).wait()

    def scat(j, p):
        pltpu.sync_copy(rows_v.at[p], acc_sh.at[dst_v.at[j]], add=True)

    for h in range(2):
        if h == 1:
            pltpu.sync_copy(src_hbm.at[wid, pl.ds(HALF, HALF)], src_v)
            pltpu.sync_copy(dst_hbm.at[wid, pl.ds(HALF, HALF)], dst_v)

        fire_g(0, 0)

        def body(i, carry):
            j = 2 * i
            fire_g(j + 1, 1)
            wait_g(j, 0)
            scat(j, 0)
            fire_g(j + 2, 0)
            wait_g(j + 1, 1)
            scat(j + 1, 1)
            return carry

        lax.fori_loop(0, HALF // 2 - 1, body, 0)
        fire_g(HALF - 1, 1)
        wait_g(HALF - 2, 0)
        scat(HALF - 2, 0)
        wait_g(HALF - 1, 1)
        scat(HALF - 1, 1)

    plsc.subcore_barrier()

    pltpu.sync_copy(
        acc_sh.at[pl.ds(zbase, ROWS_PER_TILE)],
        out_hbm.at[cid, pl.ds(zbase, ROWS_PER_TILE)],
    )


_TC_BLOCK = 2000


def _tc_body(x_ref, a0_ref, a1_ref, wt_ref, b_ref, o_ref):
    h = x_ref[...] + a0_ref[...] + a1_ref[...]
    o_ref[...] = (
        jnp.dot(h, wt_ref[...], preferred_element_type=jnp.float32) + b_ref[...]
    )


def _tc_linear(x, a0, a1, wt, b2d):
    grid = (N_NODES // _TC_BLOCK,)
    row_spec = pl.BlockSpec((_TC_BLOCK, D), lambda i: (i, 0))
    full_spec = pl.BlockSpec((D, D), lambda i: (0, 0))
    b_spec = pl.BlockSpec((1, D), lambda i: (0, 0))
    return pl.pallas_call(
        _tc_body,
        grid=grid,
        in_specs=[row_spec, row_spec, row_spec, full_spec, b_spec],
        out_specs=row_spec,
        out_shape=jax.ShapeDtypeStruct((N_NODES, D), jnp.float32),
    )(x, a0, a1, wt, b2d)


def kernel(x, edge_index, W, b):
    src = edge_index[0].astype(jnp.int32)
    dst = edge_index[1].astype(jnp.int32)
    per_w = N_EDGES // NW
    pad_w = CHUNKS * CHUNK - per_w
    pad_dst_row = N_NODES + jnp.arange(pad_w, dtype=jnp.int32)
    pad_src_row = (jnp.arange(pad_w, dtype=jnp.int32) * 41) % N_NODES
    src_p = jnp.concatenate(
        [src.reshape(NW, per_w), jnp.broadcast_to(pad_src_row, (NW, pad_w))], axis=1)
    dst_p = jnp.concatenate(
        [dst.reshape(NW, per_w), jnp.broadcast_to(pad_dst_row, (NW, pad_w))], axis=1)
    src_p = src_p.reshape(NW, CHUNKS, CHUNK)
    dst_p = dst_p.reshape(NW, CHUNKS, CHUNK)
    agg = _sc_agg(src_p, dst_p, x)
    return _tc_linear(x, agg[0], agg[1], W.T, b.reshape(1, D))
